# baseline (speedup 1.0000x reference)
import functools

import jax
import jax.numpy as jnp
from jax import lax
from jax.experimental import pallas as pl
from jax.experimental.pallas import tpu as pltpu
from jax.experimental.pallas import tpu_sc as plsc

_N_LEVELS = 16
_OFFS = [0, 4913, 17080, 46871, 126378, 331757, 856045, 1380333, 1904621,
         2428909, 2953197, 3477485, 4001773, 4526061, 5050349, 5574637, 6098925]
_RES = [16, 22, 30, 42, 58, 80, 111, 154, 212, 294, 406, 561, 776, 1072, 1482, 2048]
_P1 = -1640531535
_P2 = 805459861
_MASK = 524287
_N_DENSE = 5

_B = 131072
_NC, _NS = 2, 16
_NW = _NC * _NS
_PW = _B // _NW
_N = 256
_NSUB = _PW // _N
_NV = _N // 16
_GCH = 128
_G = 8 * _N // _GCH

_V = _OFFS[-1]
_VPAD = -(-_V // 8) * 8
_NLINES = _VPAD // 8

_f32 = jnp.float32
_i32 = jnp.int32


@functools.cache
def _build_encode_sc():
    mesh = plsc.VectorSubcoreMesh(core_axis_name="c", subcore_axis_name="s")

    @functools.partial(
        pl.kernel,
        out_type=jax.ShapeDtypeStruct((_B, 2 * _N_LEVELS), _f32),
        mesh=mesh,
        compiler_params=pltpu.CompilerParams(needs_layout_passes=False,
                                             use_tc_tiling_on_sc=False),
        scratch_types=[
            pltpu.VMEM((3 * _N,), _f32),
            pltpu.VMEM((6 * 16,), _f32),
            pltpu.VMEM((16 * _N_LEVELS,), _f32),
            pltpu.VMEM((16 * _N_LEVELS,), _i32),
            pltpu.VMEM((3 * _N,), _f32),
            pltpu.VMEM((_N,), _f32),
            pltpu.VMEM((8 * _N,), _i32),
            pltpu.VMEM((8 * _N,), _i32),
            pltpu.VMEM((8 * _N,), _f32),
            pltpu.VMEM((8 * _N, 16), _f32),
            pltpu.VMEM((_N, 2 * _N_LEVELS), _f32),
            pltpu.SemaphoreType.DMA,
        ],
    )
    def _encode_sc(xyzt_hbm, emb_hbm, aux_hbm, res_hbm, off_hbm, out_hbm,
                   xyz_v, aux_v, res_v, off_v, xn_v, val_v, idx_v, elo_v,
                   w_v, lines_v, out_v, sem):
        wid = lax.axis_index("s") * _NC + lax.axis_index("c")
        base = wid * _PW
        pltpu.sync_copy(aux_hbm, aux_v)
        pltpu.sync_copy(res_hbm, res_v)
        pltpu.sync_copy(off_hbm, off_v)
        it = lax.iota(_i32, 16)
        p1v = jnp.full((16,), _P1, _i32)
        p2v = jnp.full((16,), _P2, _i32)
        mkv = jnp.full((16,), _MASK, _i32)

        def sub_body(si, _):
            b0 = base + si * _N
            for d in range(3):
                pltpu.sync_copy(xyzt_hbm.at[d, pl.ds(b0, _N)],
                                xyz_v.at[pl.ds(d * _N, _N)])

            def norm_body(vi, _):
                ok = None
                for d in range(3):
                    mn = aux_v[pl.ds(d * 16, 16)]
                    mx = aux_v[pl.ds((3 + d) * 16, 16)]
                    p = xyz_v[pl.ds(d * _N + vi * 16, 16)]
                    x = (p - mn) / (mx - mn)
                    okd = (x >= 0.0) & (x <= 1.0)
                    ok = okd if ok is None else (ok & okd)
                    xn_v[pl.ds(d * _N + vi * 16, 16)] = jnp.clip(x, 0.0, 1.0)
                val_v[pl.ds(vi * 16, 16)] = jnp.where(ok, 1.0, 0.0).astype(_f32)
                return 0

            lax.fori_loop(0, _NV, norm_body, 0)

            def level_body(ll, _):
                resf = res_v[pl.ds(ll * 16, 16)]
                offl = off_v[pl.ds(ll * 16, 16)]
                resi = resf.astype(_i32)
                rm1 = resi - 1
                r1 = resi + 1
                r1sq = r1 * r1
                is_dense = jnp.broadcast_to(ll, (16,)) < _N_DENSE

                def comp_body(vi, _):
                    px = xn_v[pl.ds(0 * _N + vi * 16, 16)] * resf
                    py = xn_v[pl.ds(1 * _N + vi * 16, 16)] * resf
                    pz = xn_v[pl.ds(2 * _N + vi * 16, 16)] * resf
                    x0 = jnp.minimum(px.astype(_i32), rm1)
                    y0 = jnp.minimum(py.astype(_i32), rm1)
                    z0 = jnp.minimum(pz.astype(_i32), rm1)
                    fx = jnp.clip(px - x0.astype(_f32), 0.0, 1.0)
                    fy = jnp.clip(py - y0.astype(_f32), 0.0, 1.0)
                    fz = jnp.clip(pz - z0.astype(_f32), 0.0, 1.0)
                    v = val_v[pl.ds(vi * 16, 16)]
                    wx = (1.0 - fx, fx)
                    wy = (1.0 - fy, fy)
                    wz = ((1.0 - fz) * v, fz * v)
                    hy0 = y0 * p1v
                    hz0 = z0 * p2v
                    hx = (x0, x0 + 1)
                    hy = (hy0, hy0 + p1v)
                    hz = (hz0, hz0 + p2v)
                    dx0 = x0 * r1sq
                    dy0 = y0 * r1
                    dx = (dx0, dx0 + r1sq)
                    dy = (dy0, dy0 + r1)
                    dz = (z0, z0 + 1)
                    for c in range(8):
                        cx, cy, cz = c & 1, (c >> 1) & 1, (c >> 2) & 1
                        h = (hx[cx] ^ hy[cy] ^ hz[cz]) & mkv
                        dn = dx[cx] + dy[cy] + dz[cz]
                        g = jnp.where(is_dense, dn, h) + offl
                        w = wx[cx] * wy[cy] * wz[cz]
                        o = c * _N + vi * 16
                        idx_v[pl.ds(o, 16)] = lax.shift_right_logical(g, 3)
                        elo_v[pl.ds(o, 16)] = (g & 7) * 2
                        w_v[pl.ds(o, 16)] = w
                    return 0

                lax.fori_loop(0, _NV, comp_body, 0)

                def fire_body(g, _):
                    pltpu.async_copy(
                        emb_hbm.at[idx_v.at[pl.ds(g * _GCH, _GCH)]],
                        lines_v.at[pl.ds(g * _GCH, _GCH)], sem)
                    return 0

                lax.fori_loop(0, _G, fire_body, 0)

                def drain_body(g, _):
                    pltpu.make_async_copy(
                        emb_hbm.at[idx_v.at[pl.ds(g * _GCH, _GCH)]],
                        lines_v.at[pl.ds(g * _GCH, _GCH)], sem).wait()
                    return 0

                lax.fori_loop(0, _G, drain_body, 0)

                col0 = jnp.broadcast_to(ll * 2, (16,))

                def acc_body(vi, _):
                    acc0 = jnp.zeros((16,), _f32)
                    acc1 = jnp.zeros((16,), _f32)
                    for c in range(8):
                        o = c * _N + vi * 16
                        w = w_v[pl.ds(o, 16)]
                        e0 = elo_v[pl.ds(o, 16)]
                        jvec = it + o
                        r0 = plsc.load_gather(lines_v, [jvec, e0])
                        r1_ = plsc.load_gather(lines_v, [jvec, e0 + 1])
                        acc0 = acc0 + w * r0
                        acc1 = acc1 + w * r1_
                    pidx = it + vi * 16
                    plsc.store_scatter(out_v, [pidx, col0], acc0)
                    plsc.store_scatter(out_v, [pidx, col0 + 1], acc1)
                    return 0

                lax.fori_loop(0, _NV, acc_body, 0)
                return 0

            lax.fori_loop(0, _N_LEVELS, level_body, 0)
            pltpu.sync_copy(out_v, out_hbm.at[pl.ds(b0, _N)])
            return 0

        lax.fori_loop(0, _NSUB, sub_body, 0)

    return _encode_sc


def kernel(xyz, embeddings, min_xyz, max_xyz):
    pad = _VPAD - _V
    emb_lines = jnp.concatenate(
        [embeddings.astype(_f32),
         jnp.zeros((pad, 2), _f32)]).reshape(_NLINES, 16)
    xyzt = xyz.astype(_f32).T
    aux = jnp.concatenate([min_xyz.astype(_f32), max_xyz.astype(_f32)])
    auxb = jnp.broadcast_to(aux[:, None], (6, 16)).reshape(-1)
    resb = jnp.broadcast_to(jnp.array(_RES, _f32)[:, None],
                            (_N_LEVELS, 16)).reshape(-1)
    offb = jnp.broadcast_to(jnp.array(_OFFS[:_N_LEVELS], _i32)[:, None],
                            (_N_LEVELS, 16)).reshape(-1)
    return _build_encode_sc()(xyzt, emb_lines, auxb, resb, offb)


# speedup vs baseline: 4.7042x; 4.7042x over previous
import functools

import jax
import jax.numpy as jnp
from jax import lax
from jax.experimental import pallas as pl
from jax.experimental.pallas import tpu as pltpu
from jax.experimental.pallas import tpu_sc as plsc

_N_LEVELS = 16
_OFFS = [0, 4913, 17080, 46871, 126378, 331757, 856045, 1380333, 1904621,
         2428909, 2953197, 3477485, 4001773, 4526061, 5050349, 5574637, 6098925]
_RES = [16, 22, 30, 42, 58, 80, 111, 154, 212, 294, 406, 561, 776, 1072, 1482, 2048]
_P1 = -1640531535
_P2 = 805459861
_MASK = 524287
_N_DENSE = 5

_B = 131072
_NC, _NS = 2, 16
_NW = _NC * _NS
_PW = _B // _NW
_N = 256
_NSUB = _PW // _N
_NV = _N // 16
_GCH = 128
_G = 8 * _N // _GCH

_V = _OFFS[-1]
_VPAD = -(-_V // 8) * 8
_NLINES = _VPAD // 8

_f32 = jnp.float32
_i32 = jnp.int32


_LPW = 23823
_TAIL0 = 32 * _LPW
_NTAIL = _NLINES - _TAIL0
_LB = 1024
_NBULK = _LPW // _LB
_LREM = _LPW - _NBULK * _LB


@functools.cache
def _build_interleave():
    mesh = plsc.VectorSubcoreMesh(core_axis_name="c", subcore_axis_name="s")

    @functools.partial(
        pl.kernel,
        out_type=jax.ShapeDtypeStruct((_NLINES, 16), _f32),
        mesh=mesh,
        compiler_params=pltpu.CompilerParams(needs_layout_passes=False,
                                             use_tc_tiling_on_sc=False),
        scratch_types=[
            pltpu.VMEM((8 * _LB,), _f32),
            pltpu.VMEM((8 * _LB,), _f32),
            pltpu.VMEM((_LB, 16), _f32),
        ],
    )
    def _interleave_sc(embt_hbm, lines_hbm, f0_v, f1_v, o_v):
        wid = lax.axis_index("s") * _NC + lax.axis_index("c")
        base = wid * _LPW
        it = lax.iota(_i32, 16)

        def do_batch(l0, nl):
            ne = 8 * nl
            pltpu.sync_copy(embt_hbm.at[0, pl.ds(8 * l0, ne)],
                            f0_v.at[pl.ds(0, ne)])
            pltpu.sync_copy(embt_hbm.at[1, pl.ds(8 * l0, ne)],
                            f1_v.at[pl.ds(0, ne)])
            nv = -(-ne // 16)

            row_off = lax.shift_right_logical(it, 3)
            col0 = (it & 7) * 2

            def vbody(i, _):
                a = f0_v[pl.ds(i * 16, 16)]
                b = f1_v[pl.ds(i * 16, 16)]
                row = row_off + i * 2
                plsc.store_scatter(o_v, [row, col0], a)
                plsc.store_scatter(o_v, [row, col0 + 1], b)
                return 0

            lax.fori_loop(0, nv, vbody, 0)
            pltpu.sync_copy(o_v.at[pl.ds(0, nl)],
                            lines_hbm.at[pl.ds(l0, nl)])

        def bulk(bi, _):
            do_batch(base + bi * _LB, _LB)
            return 0

        lax.fori_loop(0, _NBULK, bulk, 0)
        do_batch(base + _NBULK * _LB, _LREM)

        @pl.when(wid == 0)
        def _():
            nz = 8 * _NTAIL
            def zbody(i, _):
                f0_v[pl.ds(i * 16, 16)] = jnp.zeros((16,), _f32)
                f1_v[pl.ds(i * 16, 16)] = jnp.zeros((16,), _f32)
                return 0
            lax.fori_loop(0, nz // 16, zbody, 0)
            nval = _V - 8 * _TAIL0
            pltpu.sync_copy(embt_hbm.at[0, pl.ds(8 * _TAIL0, nval)],
                            f0_v.at[pl.ds(0, nval)])
            pltpu.sync_copy(embt_hbm.at[1, pl.ds(8 * _TAIL0, nval)],
                            f1_v.at[pl.ds(0, nval)])

            row_off = lax.shift_right_logical(it, 3)
            col0 = (it & 7) * 2

            def vbody(i, _):
                a = f0_v[pl.ds(i * 16, 16)]
                b = f1_v[pl.ds(i * 16, 16)]
                row = row_off + i * 2
                plsc.store_scatter(o_v, [row, col0], a)
                plsc.store_scatter(o_v, [row, col0 + 1], b)
                return 0

            lax.fori_loop(0, nz // 16, vbody, 0)
            pltpu.sync_copy(o_v.at[pl.ds(0, _NTAIL)],
                            lines_hbm.at[pl.ds(_TAIL0, _NTAIL)])

    return _interleave_sc


@functools.cache
def _build_encode_sc():
    mesh = plsc.VectorSubcoreMesh(core_axis_name="c", subcore_axis_name="s")

    @functools.partial(
        pl.kernel,
        out_type=jax.ShapeDtypeStruct((_B, 2 * _N_LEVELS), _f32),
        mesh=mesh,
        compiler_params=pltpu.CompilerParams(needs_layout_passes=False,
                                             use_tc_tiling_on_sc=False),
        scratch_types=[
            pltpu.VMEM((3 * _N,), _f32),
            pltpu.VMEM((6 * 16,), _f32),
            pltpu.VMEM((16 * _N_LEVELS,), _f32),
            pltpu.VMEM((16 * _N_LEVELS,), _i32),
            pltpu.VMEM((3 * _N,), _f32),
            pltpu.VMEM((_N,), _f32),
            pltpu.VMEM((8 * _N,), _i32),
            pltpu.VMEM((8 * _N,), _i32),
            pltpu.VMEM((8 * _N,), _f32),
            pltpu.VMEM((8 * _N, 16), _f32),
            pltpu.VMEM((_N, 2 * _N_LEVELS), _f32),
            pltpu.SemaphoreType.DMA,
        ],
    )
    def _encode_sc(xyzt_hbm, emb_hbm, aux_hbm, res_hbm, off_hbm, out_hbm,
                   xyz_v, aux_v, res_v, off_v, xn_v, val_v, idx_v, elo_v,
                   w_v, lines_v, out_v, sem):
        wid = lax.axis_index("s") * _NC + lax.axis_index("c")
        base = wid * _PW
        pltpu.sync_copy(aux_hbm, aux_v)
        pltpu.sync_copy(res_hbm, res_v)
        pltpu.sync_copy(off_hbm, off_v)
        it = lax.iota(_i32, 16)
        p1v = jnp.full((16,), _P1, _i32)
        p2v = jnp.full((16,), _P2, _i32)
        mkv = jnp.full((16,), _MASK, _i32)

        def sub_body(si, _):
            b0 = base + si * _N
            for d in range(3):
                pltpu.sync_copy(xyzt_hbm.at[d, pl.ds(b0, _N)],
                                xyz_v.at[pl.ds(d * _N, _N)])

            def norm_body(vi, _):
                ok = None
                for d in range(3):
                    mn = aux_v[pl.ds(d * 16, 16)]
                    mx = aux_v[pl.ds((3 + d) * 16, 16)]
                    p = xyz_v[pl.ds(d * _N + vi * 16, 16)]
                    x = (p - mn) / (mx - mn)
                    okd = (x >= 0.0) & (x <= 1.0)
                    ok = okd if ok is None else (ok & okd)
                    xn_v[pl.ds(d * _N + vi * 16, 16)] = jnp.clip(x, 0.0, 1.0)
                val_v[pl.ds(vi * 16, 16)] = jnp.where(ok, 1.0, 0.0).astype(_f32)
                return 0

            lax.fori_loop(0, _NV, norm_body, 0)

            def level_body(ll, _):
                resf = res_v[pl.ds(ll * 16, 16)]
                offl = off_v[pl.ds(ll * 16, 16)]
                resi = resf.astype(_i32)
                rm1 = resi - 1
                r1 = resi + 1
                r1sq = r1 * r1
                is_dense = jnp.broadcast_to(ll, (16,)) < _N_DENSE

                def comp_body(vi, _):
                    px = xn_v[pl.ds(0 * _N + vi * 16, 16)] * resf
                    py = xn_v[pl.ds(1 * _N + vi * 16, 16)] * resf
                    pz = xn_v[pl.ds(2 * _N + vi * 16, 16)] * resf
                    x0 = jnp.minimum(px.astype(_i32), rm1)
                    y0 = jnp.minimum(py.astype(_i32), rm1)
                    z0 = jnp.minimum(pz.astype(_i32), rm1)
                    fx = jnp.clip(px - x0.astype(_f32), 0.0, 1.0)
                    fy = jnp.clip(py - y0.astype(_f32), 0.0, 1.0)
                    fz = jnp.clip(pz - z0.astype(_f32), 0.0, 1.0)
                    v = val_v[pl.ds(vi * 16, 16)]
                    wx = (1.0 - fx, fx)
                    wy = (1.0 - fy, fy)
                    wz = ((1.0 - fz) * v, fz * v)
                    hy0 = y0 * p1v
                    hz0 = z0 * p2v
                    hx = (x0, x0 + 1)
                    hy = (hy0, hy0 + p1v)
                    hz = (hz0, hz0 + p2v)
                    dx0 = x0 * r1sq
                    dy0 = y0 * r1
                    dx = (dx0, dx0 + r1sq)
                    dy = (dy0, dy0 + r1)
                    dz = (z0, z0 + 1)
                    for c in range(8):
                        cx, cy, cz = c & 1, (c >> 1) & 1, (c >> 2) & 1
                        h = (hx[cx] ^ hy[cy] ^ hz[cz]) & mkv
                        dn = dx[cx] + dy[cy] + dz[cz]
                        g = jnp.where(is_dense, dn, h) + offl
                        w = wx[cx] * wy[cy] * wz[cz]
                        o = c * _N + vi * 16
                        idx_v[pl.ds(o, 16)] = lax.shift_right_logical(g, 3)
                        elo_v[pl.ds(o, 16)] = (g & 7) * 2
                        w_v[pl.ds(o, 16)] = w
                    return 0

                lax.fori_loop(0, _NV, comp_body, 0)

                def fire_body(g, _):
                    pltpu.async_copy(
                        emb_hbm.at[idx_v.at[pl.ds(g * _GCH, _GCH)]],
                        lines_v.at[pl.ds(g * _GCH, _GCH)], sem)
                    return 0

                lax.fori_loop(0, _G, fire_body, 0)

                def drain_body(g, _):
                    pltpu.make_async_copy(
                        emb_hbm.at[idx_v.at[pl.ds(g * _GCH, _GCH)]],
                        lines_v.at[pl.ds(g * _GCH, _GCH)], sem).wait()
                    return 0

                lax.fori_loop(0, _G, drain_body, 0)

                col0 = jnp.broadcast_to(ll * 2, (16,))

                def acc_body(vi, _):
                    acc0 = jnp.zeros((16,), _f32)
                    acc1 = jnp.zeros((16,), _f32)
                    for c in range(8):
                        o = c * _N + vi * 16
                        w = w_v[pl.ds(o, 16)]
                        e0 = elo_v[pl.ds(o, 16)]
                        jvec = it + o
                        r0 = plsc.load_gather(lines_v, [jvec, e0])
                        r1_ = plsc.load_gather(lines_v, [jvec, e0 + 1])
                        acc0 = acc0 + w * r0
                        acc1 = acc1 + w * r1_
                    pidx = it + vi * 16
                    plsc.store_scatter(out_v, [pidx, col0], acc0)
                    plsc.store_scatter(out_v, [pidx, col0 + 1], acc1)
                    return 0

                lax.fori_loop(0, _NV, acc_body, 0)
                return 0

            lax.fori_loop(0, _N_LEVELS, level_body, 0)
            pltpu.sync_copy(out_v, out_hbm.at[pl.ds(b0, _N)])
            return 0

        lax.fori_loop(0, _NSUB, sub_body, 0)

    return _encode_sc


def kernel(xyz, embeddings, min_xyz, max_xyz):
    embt = embeddings.astype(_f32).T
    emb_lines = _build_interleave()(embt)
    xyzt = xyz.astype(_f32).T
    aux = jnp.concatenate([min_xyz.astype(_f32), max_xyz.astype(_f32)])
    auxb = jnp.broadcast_to(aux[:, None], (6, 16)).reshape(-1)
    resb = jnp.broadcast_to(jnp.array(_RES, _f32)[:, None],
                            (_N_LEVELS, 16)).reshape(-1)
    offb = jnp.broadcast_to(jnp.array(_OFFS[:_N_LEVELS], _i32)[:, None],
                            (_N_LEVELS, 16)).reshape(-1)
    return _build_encode_sc()(xyzt, emb_lines, auxb, resb, offb)


# speedup vs baseline: 7.6990x; 1.6366x over previous
import functools

import jax
import jax.numpy as jnp
from jax import lax
from jax.experimental import pallas as pl
from jax.experimental.pallas import tpu as pltpu
from jax.experimental.pallas import tpu_sc as plsc

_N_LEVELS = 16
_OFFS = [0, 4913, 17080, 46871, 126378, 331757, 856045, 1380333, 1904621,
         2428909, 2953197, 3477485, 4001773, 4526061, 5050349, 5574637, 6098925]
_RES = [16, 22, 30, 42, 58, 80, 111, 154, 212, 294, 406, 561, 776, 1072, 1482, 2048]
_P1 = -1640531535
_P2 = 805459861
_MASK = 524287
_N_DENSE = 5

_B = 131072
_NC, _NS = 2, 16
_NW = _NC * _NS
_PW = _B // _NW
_N = 256
_NSUB = _PW // _N
_NV = _N // 16
_GCH = 128
_G = 8 * _N // _GCH

_V = _OFFS[-1]
_VPAD = -(-_V // 8) * 8
_NLINES = _VPAD // 8

_f32 = jnp.float32
_i32 = jnp.int32


_LPW = 23823
_TAIL0 = 32 * _LPW
_NTAIL = _NLINES - _TAIL0
_LB = 1024
_NBULK = _LPW // _LB
_LREM = _LPW - _NBULK * _LB


@functools.cache
def _build_interleave():
    mesh = plsc.VectorSubcoreMesh(core_axis_name="c", subcore_axis_name="s")

    @functools.partial(
        pl.kernel,
        out_type=jax.ShapeDtypeStruct((_NLINES, 16), _f32),
        mesh=mesh,
        compiler_params=pltpu.CompilerParams(needs_layout_passes=False,
                                             use_tc_tiling_on_sc=False),
        scratch_types=[
            pltpu.VMEM((8 * _LB,), _f32),
            pltpu.VMEM((8 * _LB,), _f32),
            pltpu.VMEM((_LB, 16), _f32),
        ],
    )
    def _interleave_sc(e0_hbm, e1_hbm, lines_hbm, f0_v, f1_v, o_v):
        wid = lax.axis_index("s") * _NC + lax.axis_index("c")
        base = wid * _LPW
        it = lax.iota(_i32, 16)

        def do_batch(l0, nl):
            ne = 8 * nl
            pltpu.sync_copy(e0_hbm.at[pl.ds(8 * l0, ne)],
                            f0_v.at[pl.ds(0, ne)])
            pltpu.sync_copy(e1_hbm.at[pl.ds(8 * l0, ne)],
                            f1_v.at[pl.ds(0, ne)])
            nv = -(-ne // 16)

            row_off = lax.shift_right_logical(it, 3)
            col0 = (it & 7) * 2

            def vbody(i, _):
                a = f0_v[pl.ds(i * 16, 16)]
                b = f1_v[pl.ds(i * 16, 16)]
                row = row_off + i * 2
                plsc.store_scatter(o_v, [row, col0], a)
                plsc.store_scatter(o_v, [row, col0 + 1], b)
                return 0

            lax.fori_loop(0, nv, vbody, 0)
            pltpu.sync_copy(o_v.at[pl.ds(0, nl)],
                            lines_hbm.at[pl.ds(l0, nl)])

        def bulk(bi, _):
            do_batch(base + bi * _LB, _LB)
            return 0

        lax.fori_loop(0, _NBULK, bulk, 0)
        do_batch(base + _NBULK * _LB, _LREM)

        @pl.when(wid == 0)
        def _():
            nz = 8 * _NTAIL
            def zbody(i, _):
                f0_v[pl.ds(i * 16, 16)] = jnp.zeros((16,), _f32)
                f1_v[pl.ds(i * 16, 16)] = jnp.zeros((16,), _f32)
                return 0
            lax.fori_loop(0, nz // 16, zbody, 0)
            nval = _V - 8 * _TAIL0
            pltpu.sync_copy(e0_hbm.at[pl.ds(8 * _TAIL0, nval)],
                            f0_v.at[pl.ds(0, nval)])
            pltpu.sync_copy(e1_hbm.at[pl.ds(8 * _TAIL0, nval)],
                            f1_v.at[pl.ds(0, nval)])

            row_off = lax.shift_right_logical(it, 3)
            col0 = (it & 7) * 2

            def vbody(i, _):
                a = f0_v[pl.ds(i * 16, 16)]
                b = f1_v[pl.ds(i * 16, 16)]
                row = row_off + i * 2
                plsc.store_scatter(o_v, [row, col0], a)
                plsc.store_scatter(o_v, [row, col0 + 1], b)
                return 0

            lax.fori_loop(0, nz // 16, vbody, 0)
            pltpu.sync_copy(o_v.at[pl.ds(0, _NTAIL)],
                            lines_hbm.at[pl.ds(_TAIL0, _NTAIL)])

    return _interleave_sc


@functools.cache
def _build_encode_sc():
    mesh = plsc.VectorSubcoreMesh(core_axis_name="c", subcore_axis_name="s")

    @functools.partial(
        pl.kernel,
        out_type=jax.ShapeDtypeStruct((_B, 2 * _N_LEVELS), _f32),
        mesh=mesh,
        compiler_params=pltpu.CompilerParams(needs_layout_passes=False,
                                             use_tc_tiling_on_sc=False),
        scratch_types=[
            pltpu.VMEM((3 * _N,), _f32),
            pltpu.VMEM((6 * 16,), _f32),
            pltpu.VMEM((16 * _N_LEVELS,), _f32),
            pltpu.VMEM((16 * _N_LEVELS,), _i32),
            pltpu.VMEM((3 * _N,), _f32),
            pltpu.VMEM((_N,), _f32),
            pltpu.VMEM((8 * _N,), _i32),
            pltpu.VMEM((8 * _N,), _i32),
            pltpu.VMEM((8 * _N,), _f32),
            pltpu.VMEM((8 * _N, 16), _f32),
            pltpu.VMEM((_N, 2 * _N_LEVELS), _f32),
            pltpu.SemaphoreType.DMA,
        ],
    )
    def _encode_sc(xyzt_hbm, emb_hbm, aux_hbm, res_hbm, off_hbm, out_hbm,
                   xyz_v, aux_v, res_v, off_v, xn_v, val_v, idx_v, elo_v,
                   w_v, lines_v, out_v, sem):
        wid = lax.axis_index("s") * _NC + lax.axis_index("c")
        base = wid * _PW
        pltpu.sync_copy(aux_hbm, aux_v)
        pltpu.sync_copy(res_hbm, res_v)
        pltpu.sync_copy(off_hbm, off_v)
        it = lax.iota(_i32, 16)
        p1v = jnp.full((16,), _P1, _i32)
        p2v = jnp.full((16,), _P2, _i32)
        mkv = jnp.full((16,), _MASK, _i32)

        def sub_body(si, _):
            b0 = base + si * _N
            for d in range(3):
                pltpu.sync_copy(xyzt_hbm.at[d, pl.ds(b0, _N)],
                                xyz_v.at[pl.ds(d * _N, _N)])

            def norm_body(vi, _):
                ok = None
                for d in range(3):
                    mn = aux_v[pl.ds(d * 16, 16)]
                    mx = aux_v[pl.ds((3 + d) * 16, 16)]
                    p = xyz_v[pl.ds(d * _N + vi * 16, 16)]
                    x = (p - mn) / (mx - mn)
                    okd = (x >= 0.0) & (x <= 1.0)
                    ok = okd if ok is None else (ok & okd)
                    xn_v[pl.ds(d * _N + vi * 16, 16)] = jnp.clip(x, 0.0, 1.0)
                val_v[pl.ds(vi * 16, 16)] = jnp.where(ok, 1.0, 0.0).astype(_f32)
                return 0

            lax.fori_loop(0, _NV, norm_body, 0)

            def level_body(ll, _):
                resf = res_v[pl.ds(ll * 16, 16)]
                offl = off_v[pl.ds(ll * 16, 16)]
                resi = resf.astype(_i32)
                rm1 = resi - 1
                r1 = resi + 1
                r1sq = r1 * r1
                is_dense = jnp.broadcast_to(ll, (16,)) < _N_DENSE

                def comp_body(vi, _):
                    px = xn_v[pl.ds(0 * _N + vi * 16, 16)] * resf
                    py = xn_v[pl.ds(1 * _N + vi * 16, 16)] * resf
                    pz = xn_v[pl.ds(2 * _N + vi * 16, 16)] * resf
                    x0 = jnp.minimum(px.astype(_i32), rm1)
                    y0 = jnp.minimum(py.astype(_i32), rm1)
                    z0 = jnp.minimum(pz.astype(_i32), rm1)
                    fx = jnp.clip(px - x0.astype(_f32), 0.0, 1.0)
                    fy = jnp.clip(py - y0.astype(_f32), 0.0, 1.0)
                    fz = jnp.clip(pz - z0.astype(_f32), 0.0, 1.0)
                    v = val_v[pl.ds(vi * 16, 16)]
                    wx = (1.0 - fx, fx)
                    wy = (1.0 - fy, fy)
                    wz = ((1.0 - fz) * v, fz * v)
                    hy0 = y0 * p1v
                    hz0 = z0 * p2v
                    hx = (x0, x0 + 1)
                    hy = (hy0, hy0 + p1v)
                    hz = (hz0, hz0 + p2v)
                    dx0 = x0 * r1sq
                    dy0 = y0 * r1
                    dx = (dx0, dx0 + r1sq)
                    dy = (dy0, dy0 + r1)
                    dz = (z0, z0 + 1)
                    for c in range(8):
                        cx, cy, cz = c & 1, (c >> 1) & 1, (c >> 2) & 1
                        h = (hx[cx] ^ hy[cy] ^ hz[cz]) & mkv
                        dn = dx[cx] + dy[cy] + dz[cz]
                        g = jnp.where(is_dense, dn, h) + offl
                        w = wx[cx] * wy[cy] * wz[cz]
                        o = c * _N + vi * 16
                        idx_v[pl.ds(o, 16)] = lax.shift_right_logical(g, 3)
                        elo_v[pl.ds(o, 16)] = (g & 7) * 2
                        w_v[pl.ds(o, 16)] = w
                    return 0

                lax.fori_loop(0, _NV, comp_body, 0)

                def fire_body(g, _):
                    pltpu.async_copy(
                        emb_hbm.at[idx_v.at[pl.ds(g * _GCH, _GCH)]],
                        lines_v.at[pl.ds(g * _GCH, _GCH)], sem)
                    return 0

                lax.fori_loop(0, _G, fire_body, 0)

                def drain_body(g, _):
                    pltpu.make_async_copy(
                        emb_hbm.at[idx_v.at[pl.ds(g * _GCH, _GCH)]],
                        lines_v.at[pl.ds(g * _GCH, _GCH)], sem).wait()
                    return 0

                lax.fori_loop(0, _G, drain_body, 0)

                col0 = jnp.broadcast_to(ll * 2, (16,))

                def acc_body(vi, _):
                    acc0 = jnp.zeros((16,), _f32)
                    acc1 = jnp.zeros((16,), _f32)
                    for c in range(8):
                        o = c * _N + vi * 16
                        w = w_v[pl.ds(o, 16)]
                        e0 = elo_v[pl.ds(o, 16)]
                        jvec = it + o
                        r0 = plsc.load_gather(lines_v, [jvec, e0])
                        r1_ = plsc.load_gather(lines_v, [jvec, e0 + 1])
                        acc0 = acc0 + w * r0
                        acc1 = acc1 + w * r1_
                    pidx = it + vi * 16
                    plsc.store_scatter(out_v, [pidx, col0], acc0)
                    plsc.store_scatter(out_v, [pidx, col0 + 1], acc1)
                    return 0

                lax.fori_loop(0, _NV, acc_body, 0)
                return 0

            lax.fori_loop(0, _N_LEVELS, level_body, 0)
            pltpu.sync_copy(out_v, out_hbm.at[pl.ds(b0, _N)])
            return 0

        lax.fori_loop(0, _NSUB, sub_body, 0)

    return _encode_sc


def kernel(xyz, embeddings, min_xyz, max_xyz):
    emb = embeddings.astype(_f32)
    emb_lines = _build_interleave()(emb[:, 0], emb[:, 1])
    xyzt = xyz.astype(_f32).T
    aux = jnp.concatenate([min_xyz.astype(_f32), max_xyz.astype(_f32)])
    auxb = jnp.broadcast_to(aux[:, None], (6, 16)).reshape(-1)
    resb = jnp.broadcast_to(jnp.array(_RES, _f32)[:, None],
                            (_N_LEVELS, 16)).reshape(-1)
    offb = jnp.broadcast_to(jnp.array(_OFFS[:_N_LEVELS], _i32)[:, None],
                            (_N_LEVELS, 16)).reshape(-1)
    return _build_encode_sc()(xyzt, emb_lines, auxb, resb, offb)
